# Initial kernel scaffold; baseline (speedup 1.0000x reference)
import jax
import jax.numpy as jnp
from jax.experimental import pallas as pl


def kernel(key, time_signature, mode, danceability, energy, loudness, speechiness, acousticness, instrumentalness, liveness, valence, tempo, duration_ms, key_table, time_table, mode_table):
    raise NotImplementedError("write your pallas kernel here")


# speedup vs baseline: 3.6350x; 3.6350x over previous
import functools

import jax
import jax.numpy as jnp
from jax import lax
from jax.experimental import pallas as pl
from jax.experimental.pallas import tpu as pltpu
from jax.experimental.pallas import tpu_sc as plsc

B = 16384
D = 106
NC, NS = 2, 16
NW = NC * NS
RPW = B // NW
LANES = 16
NBATCH = RPW // LANES
TAB_ROWS = 23
EMB = 32


def _sc_body(idx_hbm, scal_hbm, tab_hbm, out_hbm, idx_v, scal_v, tab_v, outb):
    wid = lax.axis_index("c") * NS + lax.axis_index("s")
    base = wid * RPW

    pltpu.sync_copy(tab_hbm, tab_v)
    pltpu.sync_copy(idx_hbm.at[:, pl.ds(base, RPW)], idx_v)
    pltpu.sync_copy(scal_hbm.at[:, pl.ds(base, RPW)], scal_v)

    lane_iota = lax.iota(jnp.int32, LANES)

    def batch(i, carry):
        b16 = i * LANES
        k = idx_v[0, pl.ds(b16, LANES)]
        t = idx_v[1, pl.ds(b16, LANES)]
        m = idx_v[2, pl.ds(b16, LANES)]
        kidx = jnp.where((k >= -1) & (k <= 11), k + 2, 0)
        tidx = jnp.where((t >= 3) & (t <= 7), t - 2, 0) + 14
        midx = jnp.where((m >= 0) & (m <= 1), m + 1, 0) + 20
        kbase = kidx * EMB
        tbase = tidx * EMB
        mbase = midx * EMB
        rbase = (lane_iota + b16) * D
        for c in range(EMB):
            v = plsc.load_gather(tab_v, [kbase + c])
            plsc.store_scatter(outb, [rbase + c], v)
        for c in range(EMB):
            v = plsc.load_gather(tab_v, [tbase + c])
            plsc.store_scatter(outb, [rbase + (EMB + c)], v)
        for j in range(10):
            v = scal_v[j, pl.ds(b16, LANES)]
            plsc.store_scatter(outb, [rbase + (2 * EMB + j)], v)
        for c in range(EMB):
            v = plsc.load_gather(tab_v, [mbase + c])
            plsc.store_scatter(outb, [rbase + (2 * EMB + 10 + c)], v)
        return carry

    lax.fori_loop(0, NBATCH, batch, 0)

    pltpu.sync_copy(outb, out_hbm.at[pl.ds(base * D, RPW * D)])


@jax.jit
def _run(idx, scal, tab):
    mesh = plsc.VectorSubcoreMesh(core_axis_name="c", subcore_axis_name="s",
                                  num_cores=NC, num_subcores=NS)
    f = pl.kernel(
        _sc_body,
        out_type=jax.ShapeDtypeStruct((B * D,), jnp.float32),
        mesh=mesh,
        compiler_params=pltpu.CompilerParams(use_tc_tiling_on_sc=False,
                                             needs_layout_passes=False),
        scratch_types=[
            pltpu.VMEM((3, RPW), jnp.int32),
            pltpu.VMEM((10, RPW), jnp.float32),
            pltpu.VMEM((TAB_ROWS * EMB,), jnp.float32),
            pltpu.VMEM((RPW * D,), jnp.float32),
        ],
    )
    return f(idx, scal, tab).reshape(B, D)


def kernel(key, time_signature, mode, danceability, energy, loudness,
           speechiness, acousticness, instrumentalness, liveness, valence,
           tempo, duration_ms, key_table, time_table, mode_table):
    idx = jnp.stack([key.astype(jnp.int32), time_signature.astype(jnp.int32),
                     mode.astype(jnp.int32)])
    scal = jnp.stack([danceability, energy, loudness, speechiness,
                      acousticness, instrumentalness, liveness, valence,
                      tempo, duration_ms])
    tab = jnp.concatenate([key_table, time_table, mode_table]).reshape(-1)
    return _run(idx, scal, tab)


# speedup vs baseline: 3.7968x; 1.0445x over previous
import jax
import jax.numpy as jnp
from jax import lax
from jax.experimental import pallas as pl
from jax.experimental.pallas import tpu as pltpu
from jax.experimental.pallas import tpu_sc as plsc

B = 16384
D = 106
NC, NS = 2, 16
NW = NC * NS
RPW = B // NW
LANES = 16
NBATCH = RPW // LANES
TAB_ROWS = 23
EMB = 32


def _sc_body(key_hbm, time_hbm, mode_hbm, s0, s1, s2, s3, s4, s5, s6, s7,
             s8, s9, tab_hbm, out_hbm, idx_v, scal_v, tab_v, outb):
    wid = lax.axis_index("c") * NS + lax.axis_index("s")
    base = wid * RPW

    pltpu.sync_copy(tab_hbm, tab_v)
    pltpu.sync_copy(key_hbm.at[pl.ds(base, RPW)], idx_v.at[0])
    pltpu.sync_copy(time_hbm.at[pl.ds(base, RPW)], idx_v.at[1])
    pltpu.sync_copy(mode_hbm.at[pl.ds(base, RPW)], idx_v.at[2])
    for j, s in enumerate((s0, s1, s2, s3, s4, s5, s6, s7, s8, s9)):
        pltpu.sync_copy(s.at[pl.ds(base, RPW)], scal_v.at[j])

    lane_iota = lax.iota(jnp.int32, LANES)

    @plsc.parallel_loop(0, NBATCH, 1, unroll=2)
    def batch(i):
        b16 = i * LANES
        k = idx_v[0, pl.ds(b16, LANES)]
        t = idx_v[1, pl.ds(b16, LANES)]
        m = idx_v[2, pl.ds(b16, LANES)]
        kidx = jnp.where((k >= -1) & (k <= 11), k + 2, 0)
        tidx = jnp.where((t >= 3) & (t <= 7), t - 2, 0) + 14
        midx = jnp.where((m >= 0) & (m <= 1), m + 1, 0) + 20
        kbase = kidx * EMB
        tbase = tidx * EMB
        mbase = midx * EMB
        rbase = (lane_iota + b16) * D
        for c in range(EMB):
            v = plsc.load_gather(tab_v, [kbase + c])
            plsc.store_scatter(outb, [rbase + c], v)
        for c in range(EMB):
            v = plsc.load_gather(tab_v, [tbase + c])
            plsc.store_scatter(outb, [rbase + (EMB + c)], v)
        for j in range(10):
            v = scal_v[j, pl.ds(b16, LANES)]
            plsc.store_scatter(outb, [rbase + (2 * EMB + j)], v)
        for c in range(EMB):
            v = plsc.load_gather(tab_v, [mbase + c])
            plsc.store_scatter(outb, [rbase + (2 * EMB + 10 + c)], v)

    pltpu.sync_copy(outb, out_hbm.at[pl.ds(base * D, RPW * D)])


@jax.jit
def _run(key, time_signature, mode, s0, s1, s2, s3, s4, s5, s6, s7, s8, s9,
         tab):
    mesh = plsc.VectorSubcoreMesh(core_axis_name="c", subcore_axis_name="s",
                                  num_cores=NC, num_subcores=NS)
    f = pl.kernel(
        _sc_body,
        out_type=jax.ShapeDtypeStruct((B * D,), jnp.float32),
        mesh=mesh,
        compiler_params=pltpu.CompilerParams(use_tc_tiling_on_sc=False,
                                             needs_layout_passes=False),
        scratch_types=[
            pltpu.VMEM((3, RPW), jnp.int32),
            pltpu.VMEM((10, RPW), jnp.float32),
            pltpu.VMEM((TAB_ROWS * EMB,), jnp.float32),
            pltpu.VMEM((RPW * D,), jnp.float32),
        ],
    )
    return f(key, time_signature, mode, s0, s1, s2, s3, s4, s5, s6, s7, s8,
             s9, tab).reshape(B, D)


def kernel(key, time_signature, mode, danceability, energy, loudness,
           speechiness, acousticness, instrumentalness, liveness, valence,
           tempo, duration_ms, key_table, time_table, mode_table):
    tab = jnp.concatenate([key_table, time_table, mode_table]).reshape(-1)
    return _run(key.astype(jnp.int32), time_signature.astype(jnp.int32),
                mode.astype(jnp.int32), danceability, energy, loudness,
                speechiness, acousticness, instrumentalness, liveness,
                valence, tempo, duration_ms, tab)


# speedup vs baseline: 9.3911x; 2.4734x over previous
import jax
import jax.numpy as jnp
from jax import lax
from jax.experimental import pallas as pl
from jax.experimental.pallas import tpu as pltpu
from jax.experimental.pallas import tpu_sc as plsc

B = 16384
D = 106
NC, NS = 2, 16
NW = NC * NS
RPW = B // NW
LANES = 16
NBATCH = RPW // LANES
TAB_ROWS = 23
EMB = 32
TSTRIDE = 33


def _sc_body(key_hbm, time_hbm, mode_hbm, s0, s1, s2, s3, s4, s5, s6, s7,
             s8, s9, tab_hbm, out_hbm, kv, tv, mv, sv0, sv1, sv2, sv3, sv4,
             sv5, sv6, sv7, sv8, sv9, tab_v, outb):
    wid = lax.axis_index("c") * NS + lax.axis_index("s")
    base = wid * RPW
    svs = (sv0, sv1, sv2, sv3, sv4, sv5, sv6, sv7, sv8, sv9)

    pltpu.sync_copy(tab_hbm, tab_v)
    pltpu.sync_copy(key_hbm.at[pl.ds(base, RPW)], kv)
    pltpu.sync_copy(time_hbm.at[pl.ds(base, RPW)], tv)
    pltpu.sync_copy(mode_hbm.at[pl.ds(base, RPW)], mv)
    for j, s in enumerate((s0, s1, s2, s3, s4, s5, s6, s7, s8, s9)):
        pltpu.sync_copy(s.at[pl.ds(base, RPW)], svs[j])

    @plsc.parallel_loop(0, NBATCH, 1, unroll=2)
    def batch(i):
        b16 = i * LANES
        k = kv[pl.ds(b16, LANES)]
        t = tv[pl.ds(b16, LANES)]
        m = mv[pl.ds(b16, LANES)]
        kidx = jnp.where((k >= -1) & (k <= 11), k + 2, 0)
        tidx = jnp.where((t >= 3) & (t <= 7), t - 2, 0) + 14
        midx = jnp.where((m >= 0) & (m <= 1), m + 1, 0) + 20
        kbase = kidx * TSTRIDE
        tbase = tidx * TSTRIDE
        mbase = midx * TSTRIDE
        for c in range(EMB):
            outb[c, pl.ds(b16, LANES)] = plsc.load_gather(tab_v, [kbase + c])
        for c in range(EMB):
            outb[EMB + c, pl.ds(b16, LANES)] = plsc.load_gather(
                tab_v, [tbase + c])
        for j in range(10):
            outb[2 * EMB + j, pl.ds(b16, LANES)] = svs[j][pl.ds(b16, LANES)]
        for c in range(EMB):
            outb[2 * EMB + 10 + c, pl.ds(b16, LANES)] = plsc.load_gather(
                tab_v, [mbase + c])

    pltpu.sync_copy(outb, out_hbm.at[:, pl.ds(base, RPW)])


@jax.jit
def _run(key, time_signature, mode, s0, s1, s2, s3, s4, s5, s6, s7, s8, s9,
         tab):
    mesh = plsc.VectorSubcoreMesh(core_axis_name="c", subcore_axis_name="s",
                                  num_cores=NC, num_subcores=NS)
    f = pl.kernel(
        _sc_body,
        out_type=jax.ShapeDtypeStruct((D, B), jnp.float32),
        mesh=mesh,
        compiler_params=pltpu.CompilerParams(use_tc_tiling_on_sc=True,
                                             needs_layout_passes=False),
        scratch_types=(
            [pltpu.VMEM((RPW,), jnp.int32) for _ in range(3)]
            + [pltpu.VMEM((RPW,), jnp.float32) for _ in range(10)]
            + [pltpu.VMEM((TAB_ROWS * TSTRIDE,), jnp.float32),
               pltpu.VMEM((D, RPW), jnp.float32)]
        ),
    )
    return f(key, time_signature, mode, s0, s1, s2, s3, s4, s5, s6, s7, s8,
             s9, tab).T


def kernel(key, time_signature, mode, danceability, energy, loudness,
           speechiness, acousticness, instrumentalness, liveness, valence,
           tempo, duration_ms, key_table, time_table, mode_table):
    tab = jnp.concatenate([key_table, time_table, mode_table])
    tab = jnp.pad(tab, ((0, 0), (0, TSTRIDE - EMB))).reshape(-1)
    return _run(key.astype(jnp.int32), time_signature.astype(jnp.int32),
                mode.astype(jnp.int32), danceability, energy, loudness,
                speechiness, acousticness, instrumentalness, liveness,
                valence, tempo, duration_ms, tab)


# speedup vs baseline: 11.8948x; 1.2666x over previous
import jax
import jax.numpy as jnp
from jax import lax
from jax.experimental import pallas as pl
from jax.experimental.pallas import tpu as pltpu
from jax.experimental.pallas import tpu_sc as plsc

B = 16384
D = 106
NC, NS = 2, 16
NW = NC * NS
RPW = B // NW
LANES = 16
NBATCH = RPW // LANES
TAB_ROWS = 23
EMB = 32
TSTRIDE = 33


def _sc_body(key_hbm, time_hbm, mode_hbm, s0, s1, s2, s3, s4, s5, s6, s7,
             s8, s9, tab_hbm, out_hbm, kv, tv, mv, tab_v, outb, sem):
    wid = lax.axis_index("c") * NS + lax.axis_index("s")
    base = wid * RPW

    c_tab = pltpu.async_copy(tab_hbm, tab_v, sem)
    c_k = pltpu.async_copy(key_hbm.at[pl.ds(base, RPW)], kv, sem)
    c_t = pltpu.async_copy(time_hbm.at[pl.ds(base, RPW)], tv, sem)
    c_m = pltpu.async_copy(mode_hbm.at[pl.ds(base, RPW)], mv, sem)
    scal_copies = []
    for j, s in enumerate((s0, s1, s2, s3, s4, s5, s6, s7, s8, s9)):
        scal_copies.append(pltpu.async_copy(
            s.at[pl.ds(base, RPW)], outb.at[2 * EMB + j], sem))
    c_tab.wait()
    c_k.wait()
    c_t.wait()
    c_m.wait()

    @plsc.parallel_loop(0, NBATCH, 1, unroll=2)
    def batch(i):
        b16 = i * LANES
        k = kv[pl.ds(b16, LANES)]
        t = tv[pl.ds(b16, LANES)]
        m = mv[pl.ds(b16, LANES)]
        kidx = jnp.where((k >= -1) & (k <= 11), k + 2, 0)
        tidx = jnp.where((t >= 3) & (t <= 7), t - 2, 0) + 14
        midx = jnp.where((m >= 0) & (m <= 1), m + 1, 0) + 20
        kbase = kidx * TSTRIDE
        tbase = tidx * TSTRIDE
        mbase = midx * TSTRIDE
        for c in range(EMB):
            outb[c, pl.ds(b16, LANES)] = plsc.load_gather(tab_v, [kbase + c])
        for c in range(EMB):
            outb[EMB + c, pl.ds(b16, LANES)] = plsc.load_gather(
                tab_v, [tbase + c])
        for c in range(EMB):
            outb[2 * EMB + 10 + c, pl.ds(b16, LANES)] = plsc.load_gather(
                tab_v, [mbase + c])

    for c in scal_copies:
        c.wait()
    pltpu.sync_copy(outb, out_hbm.at[:, pl.ds(base, RPW)])


@jax.jit
def _run(key, time_signature, mode, s0, s1, s2, s3, s4, s5, s6, s7, s8, s9,
         tab):
    mesh = plsc.VectorSubcoreMesh(core_axis_name="c", subcore_axis_name="s",
                                  num_cores=NC, num_subcores=NS)
    f = pl.kernel(
        _sc_body,
        out_type=jax.ShapeDtypeStruct((D, B), jnp.float32),
        mesh=mesh,
        compiler_params=pltpu.CompilerParams(use_tc_tiling_on_sc=True,
                                             needs_layout_passes=False),
        scratch_types=(
            [pltpu.VMEM((RPW,), jnp.int32) for _ in range(3)]
            + [pltpu.VMEM((TAB_ROWS * TSTRIDE,), jnp.float32),
               pltpu.VMEM((D, RPW), jnp.float32),
               pltpu.SemaphoreType.DMA]
        ),
    )
    return f(key, time_signature, mode, s0, s1, s2, s3, s4, s5, s6, s7, s8,
             s9, tab).T


def kernel(key, time_signature, mode, danceability, energy, loudness,
           speechiness, acousticness, instrumentalness, liveness, valence,
           tempo, duration_ms, key_table, time_table, mode_table):
    tab = jnp.concatenate([key_table, time_table, mode_table])
    tab = jnp.pad(tab, ((0, 0), (0, TSTRIDE - EMB))).reshape(-1)
    return _run(key.astype(jnp.int32), time_signature.astype(jnp.int32),
                mode.astype(jnp.int32), danceability, energy, loudness,
                speechiness, acousticness, instrumentalness, liveness,
                valence, tempo, duration_ms, tab)
